# Initial kernel scaffold; baseline (speedup 1.0000x reference)
import jax
import jax.numpy as jnp
from jax.experimental import pallas as pl


def kernel(cls_logits, params, params_init, tgt_params, tgt_pts):
    raise NotImplementedError("write your pallas kernel here")


# speedup vs baseline: 2.2886x; 2.2886x over previous
import jax
import jax.numpy as jnp
from jax.experimental import pallas as pl
from jax.experimental.pallas import tpu as pltpu

MAX_THETA = 90.0
MAX_RADIUS = 400.0
TH_THETA = 5.0
TH_RADIUS = 20.0
W_CLS = 1.0
W_REG = 1.0
GAMMA = 2.0


def _tc_loss_body(l0, l1, pth, pra, pith, pira, tt, tr, tp0, out):
    B, N = pith.shape
    G = tt.shape[1]
    pith_v = pith[...]
    pira_v = pira[...]
    pth_v = pth[...]
    pra_v = pra[...]
    iota_n = jax.lax.broadcasted_iota(jnp.int32, (B, N), 1)
    ttn = (tt[...] + MAX_THETA) / (2.0 * MAX_THETA)
    trn = (tr[...] + MAX_RADIUS) / (2.0 * MAX_RADIUS)
    valid = tp0[...] != -1000.0

    valid_f = jnp.where(valid, 1.0, 0.0)
    gt_any = jnp.zeros((B, N), dtype=jnp.float32)
    reg_acc = jnp.zeros((B, 1), dtype=jnp.float32)
    for g in range(G):
        ttg = ttn[:, g:g + 1]
        trg = trn[:, g:g + 1]
        vg = valid_f[:, g:g + 1]
        td = jnp.abs(ttg - pith_v)
        rd = jnp.abs(trg - pira_v)
        cond = (td < TH_THETA / MAX_THETA) & (rd < TH_RADIUS / MAX_RADIUS)
        cond_f = jnp.where(cond, 1.0, 0.0)
        has_pos = jnp.max(cond_f, axis=1, keepdims=True)
        dist2 = td * td + rd * rd
        m = jnp.min(dist2, axis=1, keepdims=True)
        fb = jnp.min(jnp.where(dist2 == m, iota_n, N), axis=1, keepdims=True)
        oh_f = jnp.where(iota_n == fb, 1.0, 0.0)
        dt = ttg - pth_v
        dr = trg - pra_v
        cost = 0.5 * (dt * dt + dr * dr)
        condsum = jnp.sum(cond_f * cost, axis=1, keepdims=True)
        cost_fb = jnp.sum(oh_f * cost, axis=1, keepdims=True)
        pos_f = jnp.where(has_pos > 0.5, cond_f, oh_f) * vg
        gt_any = jnp.maximum(gt_any, pos_f)
        piece = vg * jnp.where(has_pos > 0.5, condsum, cost_fb)
        reg_acc = reg_acc + piece
    loss_reg = jnp.sum(reg_acc) / B

    l0_v = l0[...]
    l1_v = l1[...]
    mx = jnp.maximum(l0_v, l1_v)
    a0 = l0_v - mx
    a1 = l1_v - mx
    e0 = jnp.exp(a0)
    e1 = jnp.exp(a1)
    z = e0 + e1
    logz = jnp.log(z)
    s0 = e0 / z
    s1 = e1 / z
    f0 = s1 * s1
    f1 = s0 * s0
    picked = jnp.where(gt_any > 0.5, f1 * (a1 - logz), f0 * (a0 - logz))
    loss_cls = -jnp.sum(picked) / (B * N)
    out[0, 0] = W_CLS * loss_cls + W_REG * loss_reg


def kernel(cls_logits, params, params_init, tgt_params, tgt_pts):
    l0 = cls_logits[:, :, 0]
    l1 = cls_logits[:, :, 1]
    pth = params[:, :, 0]
    pra = params[:, :, 1]
    pith = params_init[:, :, 0]
    pira = params_init[:, :, 1]
    tt = tgt_params[:, :, 0]
    tr = tgt_params[:, :, 1]
    tp0 = tgt_pts[:, :, 0]
    out = pl.pallas_call(
        _tc_loss_body,
        out_shape=jax.ShapeDtypeStruct((1, 1), jnp.float32),
        in_specs=[pl.BlockSpec(memory_space=pltpu.VMEM) for _ in range(9)],
        out_specs=pl.BlockSpec(memory_space=pltpu.SMEM),
    )(l0, l1, pth, pra, pith, pira, tt, tr, tp0)
    return out[0, 0]
